# baseline (speedup 1.0000x reference)
import functools

import jax
import jax.numpy as jnp
from jax import lax
from jax.experimental import pallas as pl
from jax.experimental.pallas import tpu as pltpu
from jax.experimental.pallas import tpu_sc as plsc

VOCAB = 100000
EMBED = 64
BATCH = 16384
CTX = 20
TBLK = 2048
NBLK = -(-VOCAB // TBLK)
PROWS = NBLK * (TBLK // 2)

NUM_CORES = 2
NUM_SUBCORES = 16
NW = NUM_CORES * NUM_SUBCORES
PER_W = BATCH // NW
CH = 16
N_CHUNKS = PER_W // CH
LANES = 16


def _tc_pack(tt):
  def body(in_ref, out_ref):
    y = in_ref[...].T
    out_ref[...] = jnp.concatenate(
        [y[: TBLK // 2], y[TBLK // 2:]], axis=1)

  return pl.pallas_call(
      body,
      grid=(NBLK,),
      in_specs=[pl.BlockSpec((EMBED, TBLK), lambda j: (0, j))],
      out_specs=pl.BlockSpec((TBLK // 2, 2 * EMBED), lambda j: (j, 0)),
      out_shape=jax.ShapeDtypeStruct((PROWS, 2 * EMBED), jnp.float32),
  )(tt)


def _sc_scores(ids_t, center_ids, ctab2, gtab2):
  mesh = plsc.VectorSubcoreMesh(core_axis_name="c", subcore_axis_name="s")

  @functools.partial(
      pl.kernel,
      out_type=jax.ShapeDtypeStruct((BATCH,), jnp.float32),
      mesh=mesh,
      compiler_params=pltpu.CompilerParams(needs_layout_passes=False,
                                           use_tc_tiling_on_sc=True),
      scratch_types=[
          pltpu.VMEM((24, PER_W), jnp.int32),
          pltpu.VMEM((PER_W,), jnp.int32),
          pltpu.VMEM((CTX * CH,), jnp.int32),
          pltpu.VMEM((CTX * CH,), jnp.int32),
          pltpu.VMEM((CH,), jnp.int32),
          pltpu.VMEM((CH,), jnp.int32),
          pltpu.VMEM((CTX * CH, 2 * EMBED), jnp.float32),
          pltpu.VMEM((CTX * CH, 2 * EMBED), jnp.float32),
          pltpu.VMEM((CH, 2 * EMBED), jnp.float32),
          pltpu.VMEM((CH, 2 * EMBED), jnp.float32),
          pltpu.VMEM((1, 2 * EMBED), jnp.float32),
          pltpu.VMEM((PER_W,), jnp.float32),
          pltpu.SemaphoreType.DMA,
          pltpu.SemaphoreType.DMA,
          pltpu.SemaphoreType.DMA,
          pltpu.SemaphoreType.DMA,
      ],
  )
  def kern(ids_hbm, cids_hbm, ctab_hbm, gtab_hbm, out_hbm,
           ids_v, cids_v, idx_a, idx_b, cidx_a, cidx_b,
           rows_a, rows_b, crows_a, crows_b, row0_v, sc_v,
           sem1a, sem2a, sem1b, sem2b):
    wid = lax.axis_index("s") * NUM_CORES + lax.axis_index("c")
    wbase = wid * PER_W
    lanes = lax.iota(jnp.int32, LANES)
    ones = jnp.ones((LANES,), jnp.int32)
    zeros = jnp.zeros((LANES,), jnp.int32)
    zf = jnp.zeros((LANES,), jnp.float32)

    pltpu.sync_copy(ids_hbm.at[:, pl.ds(wbase, PER_W)], ids_v)
    pltpu.sync_copy(cids_hbm.at[pl.ds(wbase, PER_W)], cids_v)
    pltpu.sync_copy(ctab_hbm.at[pl.ds(0, 1), :], row0_v)

    bufs = [(idx_a, cidx_a, rows_a, crows_a, sem1a, sem2a),
            (idx_b, cidx_b, rows_b, crows_b, sem1b, sem2b)]

    def issue(ch, buf):
      idx, cidx, rows, crows, s1, s2 = buf
      off = ch * CH
      for j in range(CTX):
        idv = ids_v[j, pl.ds(off, LANES)]
        idx[pl.ds(j * CH, LANES)] = ((idv >> 11) << 10) + (idv & 1023)
      cidv = cids_v[pl.ds(off, LANES)]
      cidx[pl.ds(0, LANES)] = ((cidv >> 11) << 10) + (cidv & 1023)
      pltpu.async_copy(ctab_hbm.at[idx], rows, s1)
      pltpu.async_copy(gtab_hbm.at[cidx], crows, s2)

    def compute(ch, buf):
      idx, cidx, rows, crows, s1, s2 = buf
      pltpu.make_async_copy(ctab_hbm.at[idx], rows, s1).wait()
      pltpu.make_async_copy(gtab_hbm.at[cidx], crows, s2).wait()
      off = ch * CH

      rowids = []
      colbases = []
      n0 = zeros
      for j in range(CTX):
        idv = ids_v[j, pl.ds(off, LANES)]
        rowids.append(j * CH + lanes)
        colbases.append(((idv >> 10) & 1) * EMBED)
        n0 = n0 + jnp.where(idv == 0, ones, zeros)
      cidv = cids_v[pl.ds(off, LANES)]
      ccol = ((cidv >> 10) & 1) * EMBED

      def d_body(d, carry):
        dot, q = carry
        dv = jnp.full((LANES,), d, jnp.int32)
        acc = zf
        for j in range(CTX):
          acc = acc + plsc.load_gather(rows, [rowids[j], colbases[j] + d])
        cen = plsc.load_gather(crows, [lanes, ccol + d])
        r0 = plsc.load_gather(row0_v, [zeros, dv])
        return dot + acc * cen, q + r0 * cen

      dot, q = lax.fori_loop(0, EMBED, d_body, (zf, zf))
      n0f = n0.astype(jnp.float32)
      cnt = jnp.full((LANES,), jnp.float32(CTX)) - n0f
      sc_v[pl.ds(off, LANES)] = (dot - n0f * q) / cnt

    issue(0, bufs[0])

    def pair_body(i, carry):
      c0 = 2 * i
      issue(c0 + 1, bufs[1])
      compute(c0, bufs[0])

      @pl.when(i < N_CHUNKS // 2 - 1)
      def _():
        issue(c0 + 2, bufs[0])

      compute(c0 + 1, bufs[1])
      return carry

    lax.fori_loop(0, N_CHUNKS // 2, pair_body, 0)
    pltpu.sync_copy(sc_v, out_hbm.at[pl.ds(wbase, PER_W)])

  return kern(ids_t, center_ids, ctab2, gtab2)


def _tc_loss(scores, labels):
  s2 = scores.reshape(128, 128)
  y2 = labels.reshape(128, 128)

  def body(s_ref, y_ref, o_ref):
    s = s_ref[...]
    y = y_ref[...]
    p = jax.nn.sigmoid(s)
    loss = -(y * jnp.log(p + 1e-08) + (1.0 - y) * jnp.log(1.0 - p + 1e-08))
    o_ref[0, 0] = jnp.sum(loss) / jnp.float32(BATCH)

  out = pl.pallas_call(
      body,
      out_shape=jax.ShapeDtypeStruct((1, 1), jnp.float32),
      out_specs=pl.BlockSpec(memory_space=pltpu.SMEM),
  )(s2, y2)
  return out[0, 0]


@jax.jit
def kernel(context_ids, center_ids, labels, context_table, center_table):
  ids_t = jnp.pad(context_ids.astype(jnp.int32).T, ((0, 4), (0, 0)))
  cids = center_ids.astype(jnp.int32)
  ctab2 = _tc_pack(context_table.astype(jnp.float32).T)
  gtab2 = _tc_pack(center_table.astype(jnp.float32).T)
  scores = _sc_scores(ids_t, cids, ctab2, gtab2)
  return _tc_loss(scores, labels.astype(jnp.float32))


# speedup vs baseline: 2.2969x; 2.2969x over previous
import functools

import jax
import jax.numpy as jnp
from jax import lax
from jax.experimental import pallas as pl
from jax.experimental.pallas import tpu as pltpu
from jax.experimental.pallas import tpu_sc as plsc

VOCAB = 100000
EMBED = 64
BATCH = 16384
CTX = 20

NUM_CORES = 2
NUM_SUBCORES = 16
NW = NUM_CORES * NUM_SUBCORES
PER_W = BATCH // NW
CH = 32
N_CHUNKS = PER_W // CH
LANES = 16
NV = EMBED // LANES


def _sc_pool(ids_flat, context_table):
  mesh = plsc.VectorSubcoreMesh(core_axis_name="c", subcore_axis_name="s")

  @functools.partial(
      pl.kernel,
      out_type=(jax.ShapeDtypeStruct((BATCH, EMBED), jnp.float32),
                jax.ShapeDtypeStruct((BATCH,), jnp.float32)),
      mesh=mesh,
      compiler_params=pltpu.CompilerParams(needs_layout_passes=False,
                                           use_tc_tiling_on_sc=False),
      scratch_types=[
          pltpu.VMEM((2, CH * CTX), jnp.int32),
          pltpu.VMEM((2, CH * CTX, EMBED), jnp.float32),
          pltpu.VMEM((EMBED,), jnp.float32),
          pltpu.VMEM((CH, EMBED), jnp.float32),
          pltpu.VMEM((PER_W,), jnp.float32),
          pltpu.SemaphoreType.DMA,
          pltpu.SemaphoreType.DMA,
      ],
  )
  def kern(ids_hbm, ctab_hbm, sums_hbm, cnt_hbm,
           idx_v, rows_v, row0_v, sum_v, cn_v, sem1a, sem1b):
    wid = lax.axis_index("s") * NUM_CORES + lax.axis_index("c")
    wbase = wid * PER_W
    lanes = lax.iota(jnp.int32, LANES)
    ones = jnp.ones((LANES,), jnp.int32)
    zeros = jnp.zeros((LANES,), jnp.int32)

    pltpu.sync_copy(ctab_hbm.at[0], row0_v)
    row0 = tuple(row0_v[pl.ds(v * LANES, LANES)] for v in range(NV))

    bufs = [(idx_v.at[0], rows_v.at[0], sem1a),
            (idx_v.at[1], rows_v.at[1], sem1b)]

    def issue(c, buf):
      idx, rows, s1 = buf
      ebase = wbase + c * CH
      pltpu.sync_copy(ids_hbm.at[pl.ds(ebase * CTX, CH * CTX)], idx)
      pltpu.async_copy(ctab_hbm.at[idx], rows, s1)

    def compute(c, buf):
      idx, rows, s1 = buf
      pltpu.make_async_copy(ctab_hbm.at[idx], rows, s1).wait()
      ebase = wbase + c * CH

      def g_body(g, carry):
        def e_body(k, cvec):
          e = g * LANES + k

          accs = [jnp.zeros((LANES,), jnp.float32) for _ in range(NV)]
          for j in range(CTX):
            r = e * CTX + j
            for v in range(NV):
              accs[v] = accs[v] + rows[r, pl.ds(v * LANES, LANES)]

          v1 = idx[pl.ds(e * CTX, LANES)]
          v2 = idx[pl.ds(e * CTX + 4, LANES)]
          nz = (jnp.where(v1 == 0, ones, zeros)
                + jnp.where((v2 == 0) & (lanes >= 12), ones, zeros))
          n0f = jnp.sum(nz).astype(jnp.float32)

          for v in range(NV):
            sum_v[e, pl.ds(v * LANES, LANES)] = accs[v] - n0f * row0[v]
          cnt = jnp.float32(CTX) - n0f
          return jnp.where(lanes == k, jnp.full((LANES,), cnt), cvec)

        cvec = lax.fori_loop(0, LANES, e_body,
                             jnp.zeros((LANES,), jnp.float32))
        cn_v[pl.ds(c * CH + g * LANES, LANES)] = cvec
        return carry

      lax.fori_loop(0, CH // LANES, g_body, 0)
      pltpu.sync_copy(sum_v, sums_hbm.at[pl.ds(ebase, CH), :])

    issue(0, bufs[0])

    def pair_body(i, carry):
      c0 = 2 * i
      issue(c0 + 1, bufs[1])
      compute(c0, bufs[0])

      @pl.when(i < N_CHUNKS // 2 - 1)
      def _():
        issue(c0 + 2, bufs[0])

      compute(c0 + 1, bufs[1])
      return carry

    lax.fori_loop(0, N_CHUNKS // 2, pair_body, 0)
    pltpu.sync_copy(cn_v, cnt_hbm.at[pl.ds(wbase, PER_W)])

  return kern(ids_flat, context_table)


def _sc_dot(sums, cnt, center_ids, center_table):
  mesh = plsc.VectorSubcoreMesh(core_axis_name="c", subcore_axis_name="s")

  @functools.partial(
      pl.kernel,
      out_type=jax.ShapeDtypeStruct((BATCH,), jnp.float32),
      mesh=mesh,
      compiler_params=pltpu.CompilerParams(needs_layout_passes=False,
                                           use_tc_tiling_on_sc=False),
      scratch_types=[
          pltpu.VMEM((2, CH), jnp.int32),
          pltpu.VMEM((2, CH, EMBED), jnp.float32),
          pltpu.VMEM((2, CH, EMBED), jnp.float32),
          pltpu.VMEM((2, CH), jnp.float32),
          pltpu.VMEM((PER_W,), jnp.float32),
          pltpu.SemaphoreType.DMA,
          pltpu.SemaphoreType.DMA,
          pltpu.SemaphoreType.DMA,
          pltpu.SemaphoreType.DMA,
      ],
  )
  def kern(sums_hbm, cnt_hbm, cids_hbm, gtab_hbm, out_hbm,
           cidx_v, crows_v, sums_v, cn_v, sc_v, sa, sb, ta, tb):
    wid = lax.axis_index("s") * NUM_CORES + lax.axis_index("c")
    wbase = wid * PER_W
    lanes = lax.iota(jnp.int32, LANES)

    bufs = [(cidx_v.at[0], crows_v.at[0], sums_v.at[0], cn_v.at[0], sa, ta),
            (cidx_v.at[1], crows_v.at[1], sums_v.at[1], cn_v.at[1], sb, tb)]

    def issue(c, buf):
      cidx, crows, sm, cn, s1, s2 = buf
      ebase = wbase + c * CH
      pltpu.sync_copy(cids_hbm.at[pl.ds(ebase, CH)], cidx)
      pltpu.sync_copy(cnt_hbm.at[pl.ds(ebase, CH)], cn)
      pltpu.async_copy(gtab_hbm.at[cidx], crows, s1)
      pltpu.async_copy(sums_hbm.at[pl.ds(ebase, CH), :], sm, s2)

    def compute(c, buf):
      cidx, crows, sm, cn, s1, s2 = buf
      pltpu.make_async_copy(gtab_hbm.at[cidx], crows, s1).wait()
      ebase = wbase + c * CH
      pltpu.make_async_copy(sums_hbm.at[pl.ds(ebase, CH), :], sm, s2).wait()

      def g_body(g, carry):
        def e_body(k, dvec):
          e = g * LANES + k
          t = jnp.zeros((LANES,), jnp.float32)
          for v in range(NV):
            t = t + (sm[e, pl.ds(v * LANES, LANES)]
                     * crows[e, pl.ds(v * LANES, LANES)])
          d = jnp.sum(t)
          return jnp.where(lanes == k, jnp.full((LANES,), d), dvec)

        dvec = lax.fori_loop(0, LANES, e_body,
                             jnp.zeros((LANES,), jnp.float32))
        cvec = cn[pl.ds(g * LANES, LANES)]
        sc_v[pl.ds(c * CH + g * LANES, LANES)] = dvec / cvec
        return carry

      lax.fori_loop(0, CH // LANES, g_body, 0)

    issue(0, bufs[0])

    def pair_body(i, carry):
      c0 = 2 * i
      issue(c0 + 1, bufs[1])
      compute(c0, bufs[0])

      @pl.when(i < N_CHUNKS // 2 - 1)
      def _():
        issue(c0 + 2, bufs[0])

      compute(c0 + 1, bufs[1])
      return carry

    lax.fori_loop(0, N_CHUNKS // 2, pair_body, 0)
    pltpu.sync_copy(sc_v, out_hbm.at[pl.ds(wbase, PER_W)])

  return kern(sums, cnt, center_ids, center_table)


def _tc_loss(scores, labels):
  s2 = scores.reshape(128, 128)
  y2 = labels.reshape(128, 128)

  def body(s_ref, y_ref, o_ref):
    s = s_ref[...]
    y = y_ref[...]
    p = jax.nn.sigmoid(s)
    loss = -(y * jnp.log(p + 1e-08) + (1.0 - y) * jnp.log(1.0 - p + 1e-08))
    o_ref[0, 0] = jnp.sum(loss) / jnp.float32(BATCH)

  out = pl.pallas_call(
      body,
      out_shape=jax.ShapeDtypeStruct((1, 1), jnp.float32),
      out_specs=pl.BlockSpec(memory_space=pltpu.SMEM),
  )(s2, y2)
  return out[0, 0]


@jax.jit
def kernel(context_ids, center_ids, labels, context_table, center_table):
  ids = context_ids.astype(jnp.int32).reshape(BATCH * CTX)
  cids = center_ids.astype(jnp.int32)
  sums, cnt = _sc_pool(ids, context_table.astype(jnp.float32))
  scores = _sc_dot(sums, cnt, cids, center_table.astype(jnp.float32))
  return _tc_loss(scores, labels.astype(jnp.float32))
